# baseline (speedup 1.0000x reference)
import functools

import jax
import jax.numpy as jnp
from jax import lax
from jax.experimental import pallas as pl
from jax.experimental.pallas import tpu as pltpu
from jax.experimental.pallas import tpu_sc as plsc

N = 10000
E = 320000
D = 128
HH = 128
RW = 256
NADD = 9
TW = 8
XW = 16

NC = 2
NS = 16
NW = NC * NS
EPW = E // NW
CH = 128
NFULL = EPW // CH
TAIL = EPW - NFULL * CH

NPT = N // NS
ZB = 125

_HI = lax.Precision.DEFAULT
_F32 = jnp.float32


def _psi(v):
    return jnp.sign(v) * jnp.log1p(jnp.abs(v))


_BN_PREP = 2000


def _prep_body(h_ref, x_ref, w1a_ref, w1b_ref, rp_ref, rq_ref):
    hb = h_ref[...]
    xb = x_ref[...]
    p = jnp.dot(hb, w1a_ref[...], preferred_element_type=_F32,
                precision=_HI)
    q = jnp.dot(hb, w1b_ref[...], preferred_element_type=_F32,
                precision=_HI)
    n2 = (xb[:, 0:1] ** 2 - xb[:, 1:2] ** 2 - xb[:, 2:3] ** 2
          - xb[:, 3:4] ** 2)
    pad = jnp.zeros((xb.shape[0], RW - HH - 5), _F32)
    rp_ref[...] = jnp.concatenate([p, xb, n2, pad], axis=1)
    rq_ref[...] = jnp.concatenate([q, -xb, n2, pad], axis=1)


def _prep(h, x, w1a, w1b):
    nblk = N // _BN_PREP
    return pl.pallas_call(
        _prep_body,
        grid=(nblk,),
        in_specs=[
            pl.BlockSpec((_BN_PREP, D), lambda n: (n, 0)),
            pl.BlockSpec((_BN_PREP, 4), lambda n: (n, 0)),
            pl.BlockSpec((D, HH), lambda n: (0, 0)),
            pl.BlockSpec((D, HH), lambda n: (0, 0)),
        ],
        out_specs=[
            pl.BlockSpec((_BN_PREP, RW), lambda n: (n, 0)),
            pl.BlockSpec((_BN_PREP, RW), lambda n: (n, 0)),
        ],
        out_shape=[
            jax.ShapeDtypeStruct((N, RW), _F32),
            jax.ShapeDtypeStruct((N, RW), _F32),
        ],
    )(h, x, w1a, w1b)


def _sc_gather(rptab, rqtab, ivec, jvec):
    mesh = plsc.VectorSubcoreMesh(core_axis_name="c", subcore_axis_name="s",
                                  num_cores=NC, num_subcores=NS)

    @functools.partial(
        pl.kernel,
        out_type=jax.ShapeDtypeStruct((E, RW), _F32),
        mesh=mesh,
        scratch_types=[
            pltpu.VMEM((CH,), jnp.int32),
            pltpu.VMEM((CH,), jnp.int32),
            pltpu.VMEM((CH, RW), _F32),
            pltpu.VMEM((CH, RW), _F32),
            pltpu.VMEM((TAIL,), jnp.int32),
            pltpu.VMEM((TAIL,), jnp.int32),
            pltpu.VMEM((TAIL, RW), _F32),
            pltpu.VMEM((TAIL, RW), _F32),
            pltpu.SemaphoreType.DMA,
            pltpu.SemaphoreType.DMA,
        ],
    )
    def gather_kernel(rp_hbm, rq_hbm, i_hbm, j_hbm, s_hbm,
                      iv, jv, bp, bq, ivt, jvt, bpt, bqt, semp, semq):
        wid = lax.axis_index("s") * NC + lax.axis_index("c")
        base = wid * EPW

        @pl.loop(0, NFULL)
        def _chunk(k):
            off = base + k * CH
            pltpu.sync_copy(i_hbm.at[pl.ds(off, CH)], iv)
            pltpu.sync_copy(j_hbm.at[pl.ds(off, CH)], jv)
            cp = pltpu.async_copy(rp_hbm.at[iv], bp, semp)
            cq = pltpu.async_copy(rq_hbm.at[jv], bq, semq)
            cp.wait()
            cq.wait()

            @pl.loop(0, CH)
            def _row(r):
                for c in range(NADD):
                    sl = pl.ds(c * 16, 16)
                    bp[r, sl] = bp[r, sl] + bq[r, sl]

            pltpu.sync_copy(bp, s_hbm.at[pl.ds(off, CH)])

        offt = base + NFULL * CH
        pltpu.sync_copy(i_hbm.at[pl.ds(offt, TAIL)], ivt)
        pltpu.sync_copy(j_hbm.at[pl.ds(offt, TAIL)], jvt)
        cp = pltpu.async_copy(rp_hbm.at[ivt], bpt, semp)
        cq = pltpu.async_copy(rq_hbm.at[jvt], bqt, semq)
        cp.wait()
        cq.wait()

        @pl.loop(0, TAIL)
        def _rowt(r):
            for c in range(NADD):
                sl = pl.ds(c * 16, 16)
                bpt[r, sl] = bpt[r, sl] + bqt[r, sl]

        pltpu.sync_copy(bpt, s_hbm.at[pl.ds(offt, TAIL)])

    return gather_kernel(rptab, rqtab, ivec, jvec)


def _edge_out(s, wn, wd):
    s0 = s[:, 0:HH]
    xd = s[:, HH:HH + 4]
    n2s = s[:, HH + 4:HH + 5]
    norms = (xd[:, 0:1] ** 2 - xd[:, 1:2] ** 2 - xd[:, 2:3] ** 2
             - xd[:, 3:4] ** 2)
    dots = 0.5 * (n2s - norms)
    out = s0 + _psi(norms) * wn + _psi(dots) * wd
    return out, xd


_BE1 = 2560


def _stats_body(s_ref, wn_ref, wd_ref, ge_ref, be_ref, ab_ref, acc_ref):
    pid = pl.program_id(0)

    @pl.when(pid == 0)
    def _():
        acc_ref[...] = jnp.zeros_like(acc_ref)

    out, _ = _edge_out(s_ref[...], wn_ref[...], wd_ref[...])
    acc_ref[0:1, :] += jnp.sum(out, axis=0, keepdims=True)
    acc_ref[1:2, :] += jnp.sum(out * out, axis=0, keepdims=True)

    @pl.when(pid == pl.num_programs(0) - 1)
    def _():
        mu = acc_ref[0:1, :] * (1.0 / E)
        var = acc_ref[1:2, :] * (1.0 / E) - mu * mu
        a = ge_ref[...] * lax.rsqrt(var + 1e-5)
        b = be_ref[...] - mu * a
        ab_ref[...] = jnp.concatenate([a, b], axis=0)


def _edge_stats(s, wn, wd, ge, be):
    nblk = E // _BE1
    return pl.pallas_call(
        _stats_body,
        grid=(nblk,),
        in_specs=[
            pl.BlockSpec((_BE1, RW), lambda e: (e, 0)),
            pl.BlockSpec((1, HH), lambda e: (0, 0)),
            pl.BlockSpec((1, HH), lambda e: (0, 0)),
            pl.BlockSpec((1, HH), lambda e: (0, 0)),
            pl.BlockSpec((1, HH), lambda e: (0, 0)),
        ],
        out_specs=pl.BlockSpec((2, HH), lambda e: (0, 0)),
        out_shape=jax.ShapeDtypeStruct((2, HH), _F32),
        scratch_shapes=[pltpu.VMEM((2, HH), _F32)],
    )(s, wn, wd, ge, be)


_BE2 = 2560


def _mlp_body(s_ref, wn_ref, wd_ref, ab_ref,
              we2_ref, be2_ref, wmt_ref, bm_ref, wx1_ref, bx1_ref, wx2t_ref,
              m_ref, t16_ref):
    out, xd = _edge_out(s_ref[...], wn_ref[...], wd_ref[...])
    out = jnp.maximum(out * ab_ref[0:1, :] + ab_ref[1:2, :], 0.0)
    out2 = jnp.maximum(
        jnp.dot(out, we2_ref[...], preferred_element_type=_F32,
                precision=_HI) + be2_ref[...], 0.0)
    logit = jnp.sum(out2 * wmt_ref[...], axis=1, keepdims=True) + bm_ref[0, 0]
    w = jax.nn.sigmoid(logit)
    m = out2 * w
    m_ref[...] = m
    t = jnp.maximum(
        jnp.dot(m, wx1_ref[...], preferred_element_type=_F32,
                precision=_HI) + bx1_ref[...], 0.0)
    xm = jnp.sum(t * wx2t_ref[...], axis=1, keepdims=True)
    trans = jnp.clip(xd * xm, -100.0, 100.0)
    nrow = trans.shape[0]
    t16_ref[...] = jnp.concatenate(
        [trans, jnp.ones((nrow, 1), _F32), jnp.zeros((nrow, XW - 5), _F32)],
        axis=1)


def _edge_mlp(s, wn, wd, ab, we2, be2, wmt, bm, wx1, bx1, wx2t):
    nblk = E // _BE2
    return pl.pallas_call(
        _mlp_body,
        grid=(nblk,),
        in_specs=[
            pl.BlockSpec((_BE2, RW), lambda e: (e, 0)),
            pl.BlockSpec((1, HH), lambda e: (0, 0)),
            pl.BlockSpec((1, HH), lambda e: (0, 0)),
            pl.BlockSpec((2, HH), lambda e: (0, 0)),
            pl.BlockSpec((HH, HH), lambda e: (0, 0)),
            pl.BlockSpec((1, HH), lambda e: (0, 0)),
            pl.BlockSpec((1, HH), lambda e: (0, 0)),
            pl.BlockSpec((1, 1), lambda e: (0, 0)),
            pl.BlockSpec((HH, HH), lambda e: (0, 0)),
            pl.BlockSpec((1, HH), lambda e: (0, 0)),
            pl.BlockSpec((1, HH), lambda e: (0, 0)),
        ],
        out_specs=[
            pl.BlockSpec((_BE2, HH), lambda e: (e, 0)),
            pl.BlockSpec((_BE2, XW), lambda e: (e, 0)),
        ],
        out_shape=[
            jax.ShapeDtypeStruct((E, HH), _F32),
            jax.ShapeDtypeStruct((E, XW), _F32),
        ],
    )(s, wn, wd, ab, we2, be2, wmt, bm, wx1, bx1, wx2t)


_NSTRIPE = N // 16
_NROUND = _NSTRIPE // NS
_NXTRA = _NSTRIPE - _NROUND * NS


def _sc_scatter(rows_hbm_width, inflate):
    mesh = plsc.VectorSubcoreMesh(core_axis_name="c", subcore_axis_name="s",
                                  num_cores=NC, num_subcores=NS)
    width = rows_hbm_width

    scratch = [
        pltpu.MemorySpace.VMEM_SHARED((N, HH), _F32),
        pltpu.VMEM((CH,), jnp.int32),
        pltpu.VMEM((CH, width), _F32),
        pltpu.VMEM((TAIL,), jnp.int32),
        pltpu.VMEM((TAIL, width), _F32),
    ]
    if inflate:
        scratch += [pltpu.VMEM((CH, HH), _F32), pltpu.VMEM((TAIL, HH), _F32)]

    @functools.partial(
        pl.kernel,
        out_type=jax.ShapeDtypeStruct((NC, N, HH), _F32),
        mesh=mesh,
        scratch_types=scratch,
    )
    def scatter_kernel(rows_hbm, i_hbm, acc_hbm, acc_sh, iv, rb, ivt, rbt,
                       *inf):
        cid = lax.axis_index("c")
        sid = lax.axis_index("s")
        wid = sid * NC + cid
        zv = jnp.zeros((16,), _F32)
        zbuf = inf[0] if inflate else rb

        @pl.loop(0, 16)
        def _zm(r):
            for c in range(HH // 16):
                zbuf[r, pl.ds(c * 16, 16)] = zv

        @pl.loop(0, _NROUND)
        def _clear(k):
            off = (k * NS + sid) * 16
            pltpu.sync_copy(zbuf.at[pl.ds(0, 16)], acc_sh.at[pl.ds(off, 16)])

        @pl.when(sid < _NXTRA)
        def _():
            off = (_NROUND * NS + sid) * 16
            pltpu.sync_copy(zbuf.at[pl.ds(0, 16)], acc_sh.at[pl.ds(off, 16)])

        if inflate:
            @pl.loop(16, CH)
            def _zrest(r):
                for c in range(HH // 16):
                    inf[0][r, pl.ds(c * 16, 16)] = zv

            @pl.loop(0, TAIL)
            def _zrt(r):
                for c in range(HH // 16):
                    inf[1][r, pl.ds(c * 16, 16)] = zv

        plsc.subcore_barrier()

        base = wid * EPW

        def _do_chunk(off, n, ivb, rbb, ibuf):
            pltpu.sync_copy(i_hbm.at[pl.ds(off, n)], ivb)
            pltpu.sync_copy(rows_hbm.at[pl.ds(off, n)], rbb)
            if inflate:
                @pl.loop(0, n)
                def _inf(r):
                    ibuf[r, pl.ds(0, 16)] = rbb[r, pl.ds(0, 16)]

                pltpu.sync_copy(ibuf, acc_sh.at[ivb], add=True)
            else:
                pltpu.sync_copy(rbb, acc_sh.at[ivb], add=True)

        @pl.loop(0, NFULL)
        def _chunk(k):
            _do_chunk(base + k * CH, CH, iv, rb, inf[0] if inflate else None)

        _do_chunk(base + NFULL * CH, TAIL, ivt, rbt,
                  inf[1] if inflate else None)

        plsc.subcore_barrier()

        @pl.loop(0, _NROUND)
        def _flush(k):
            off = (k * NS + sid) * 16
            pltpu.sync_copy(acc_sh.at[pl.ds(off, 16)],
                            acc_hbm.at[cid, pl.ds(off, 16)])

        @pl.when(sid < _NXTRA)
        def _():
            off = (_NROUND * NS + sid) * 16
            pltpu.sync_copy(acc_sh.at[pl.ds(off, 16)],
                            acc_hbm.at[cid, pl.ds(off, 16)])

    return scatter_kernel


_BN_NODE = 2000


def _node1_body(h_ref, aggp_ref, na_ref, w1_ref, w2_ref, wc_ref, bh1_ref,
                gh_ref, bh_ref, hh_ref, ab2_ref, acc_ref):
    pid = pl.program_id(0)

    @pl.when(pid == 0)
    def _():
        acc_ref[...] = jnp.zeros_like(acc_ref)

    agg = aggp_ref[0] + aggp_ref[1]
    na = na_ref[...]
    hh = (jnp.dot(h_ref[...], w1_ref[...], preferred_element_type=_F32,
                  precision=_HI)
          + jnp.dot(agg, w2_ref[...], preferred_element_type=_F32,
                    precision=_HI)
          + na[:, 0:1] * wc_ref[0:1, :] + na[:, 1:2] * wc_ref[1:2, :]
          + bh1_ref[...])
    hh_ref[...] = hh
    acc_ref[0:1, :] += jnp.sum(hh, axis=0, keepdims=True)
    acc_ref[1:2, :] += jnp.sum(hh * hh, axis=0, keepdims=True)

    @pl.when(pid == pl.num_programs(0) - 1)
    def _():
        mu = acc_ref[0:1, :] * (1.0 / N)
        var = acc_ref[1:2, :] * (1.0 / N) - mu * mu
        a = gh_ref[...] * lax.rsqrt(var + 1e-5)
        b = bh_ref[...] - mu * a
        ab2_ref[...] = jnp.concatenate([a, b], axis=0)


def _node1(h, aggp, node_attr, w1, w2, wc, bh1, gh, bh):
    nblk = N // _BN_NODE
    return pl.pallas_call(
        _node1_body,
        grid=(nblk,),
        in_specs=[
            pl.BlockSpec((_BN_NODE, D), lambda n: (n, 0)),
            pl.BlockSpec((NC, _BN_NODE, HH), lambda n: (0, n, 0)),
            pl.BlockSpec((_BN_NODE, 2), lambda n: (n, 0)),
            pl.BlockSpec((D, HH), lambda n: (0, 0)),
            pl.BlockSpec((HH, HH), lambda n: (0, 0)),
            pl.BlockSpec((2, HH), lambda n: (0, 0)),
            pl.BlockSpec((1, HH), lambda n: (0, 0)),
            pl.BlockSpec((1, HH), lambda n: (0, 0)),
            pl.BlockSpec((1, HH), lambda n: (0, 0)),
        ],
        out_specs=[
            pl.BlockSpec((_BN_NODE, HH), lambda n: (n, 0)),
            pl.BlockSpec((2, HH), lambda n: (0, 0)),
        ],
        out_shape=[
            jax.ShapeDtypeStruct((N, HH), _F32),
            jax.ShapeDtypeStruct((2, HH), _F32),
        ],
        scratch_shapes=[pltpu.VMEM((2, HH), _F32)],
    )(h, aggp, node_attr, w1, w2, wc, bh1, gh, bh)


def _node2_body(hh_ref, ab2_ref, h_ref, wh2_ref, bh2_ref, x_ref, tsp_ref,
                hn_ref, xn_ref):
    hh = jnp.maximum(hh_ref[...] * ab2_ref[0:1, :] + ab2_ref[1:2, :], 0.0)
    hn_ref[...] = (h_ref[...]
                   + jnp.dot(hh, wh2_ref[...], preferred_element_type=_F32,
                             precision=_HI) + bh2_ref[...])
    ts = tsp_ref[0] + tsp_ref[1]
    cnt = jnp.maximum(ts[:, 4:5], 1.0)
    xn_ref[...] = x_ref[...] + ts[:, 0:4] / cnt


def _node2(hh, ab2, h, wh2, bh2, x, tsp):
    nblk = N // _BN_NODE
    return pl.pallas_call(
        _node2_body,
        grid=(nblk,),
        in_specs=[
            pl.BlockSpec((_BN_NODE, HH), lambda n: (n, 0)),
            pl.BlockSpec((2, HH), lambda n: (0, 0)),
            pl.BlockSpec((_BN_NODE, D), lambda n: (n, 0)),
            pl.BlockSpec((HH, D), lambda n: (0, 0)),
            pl.BlockSpec((1, D), lambda n: (0, 0)),
            pl.BlockSpec((_BN_NODE, 4), lambda n: (n, 0)),
            pl.BlockSpec((NC, _BN_NODE, HH), lambda n: (0, n, 0)),
        ],
        out_specs=[
            pl.BlockSpec((_BN_NODE, D), lambda n: (n, 0)),
            pl.BlockSpec((_BN_NODE, 4), lambda n: (n, 0)),
        ],
        out_shape=[
            jax.ShapeDtypeStruct((N, D), _F32),
            jax.ShapeDtypeStruct((N, 4), _F32),
        ],
    )(hh, ab2, h, wh2, bh2, x, tsp)


def kernel(h, x, edges, node_attr, we1, ge, be, we2, be2, wm, bm, wh1, bh1,
           gh, bh, wh2, bh2, wx1, bx1, wx2):
    ivec = edges[0]
    jvec = edges[1]
    w1a = we1[:D]
    w1b = we1[D:2 * D]
    wn = we1[2 * D].reshape(1, HH)
    wd = we1[2 * D + 1].reshape(1, HH)

    rptab, rqtab = _prep(h, x, w1a, w1b)
    s = _sc_gather(rptab, rqtab, ivec, jvec)
    ab = _edge_stats(s, wn, wd, ge.reshape(1, HH), be.reshape(1, HH))
    m, t16 = _edge_mlp(s, wn, wd, ab, we2,
                       be2.reshape(1, HH), wm.reshape(1, HH),
                       bm.reshape(1, 1), wx1, bx1.reshape(1, HH),
                       wx2.reshape(1, HH))
    aggp = _sc_scatter(HH, False)(m, ivec)
    tsp = _sc_scatter(XW, True)(t16, ivec)
    hhpre, ab2 = _node1(h, aggp, node_attr, wh1[:D], wh1[D:2 * D],
                        wh1[2 * D:], bh1.reshape(1, HH), gh.reshape(1, HH),
                        bh.reshape(1, HH))
    h_new, x_new = _node2(hhpre, ab2, h, wh2, bh2.reshape(1, D), x, tsp)
    return (h_new, x_new, m)
